# Initial kernel scaffold; baseline (speedup 1.0000x reference)
import jax
import jax.numpy as jnp
from jax.experimental import pallas as pl


def kernel(node_feat, edge_index, edge_feat, W, b, gamma, beta):
    raise NotImplementedError("write your pallas kernel here")


# speedup vs baseline: 2.2567x; 2.2567x over previous
import functools

import jax
import jax.numpy as jnp
from jax import lax
from jax.experimental import pallas as pl
from jax.experimental.pallas import tpu as pltpu
from jax.experimental.pallas import tpu_sc as plsc

N = 10000
E = 160000
D = 256
H = 128
NS = 16
EPT = E // NS
C = 80
NCHUNK = EPT // C
RPT = 624


def _sc_aggregate(nf2, src, dst, edge_feat):
    mesh = plsc.VectorSubcoreMesh(core_axis_name="c", subcore_axis_name="s")

    @functools.partial(
        pl.kernel,
        mesh=mesh,
        out_type=jax.ShapeDtypeStruct((2 * N, H), jnp.float32),
        scratch_types=[
            pltpu.VMEM((C,), jnp.int32),
            pltpu.VMEM((C,), jnp.int32),
            pltpu.VMEM((C, H), jnp.float32),
            pltpu.VMEM((C, H), jnp.float32),
            pltpu.VMEM_SHARED((N, H), jnp.float32),
            pltpu.SemaphoreType.DMA,
        ],
    )
    def k(nf2_hbm, src_hbm, dst_hbm, ef_hbm, out_hbm,
          sidx, didx, rows, efv, acc, sem):
        c = lax.axis_index("c")
        s = lax.axis_index("s")
        pltpu.sync_copy(nf2_hbm.at[pl.ds(c * N + s * RPT, RPT)],
                        acc.at[pl.ds(s * RPT, RPT)])

        @pl.when(s == 0)
        def _():
            pltpu.sync_copy(nf2_hbm.at[pl.ds(c * N + NS * RPT, N - NS * RPT)],
                            acc.at[pl.ds(NS * RPT, N - NS * RPT)])

        plsc.subcore_barrier()

        base = s * EPT

        def chunk(j, carry):
            off = base + j * C
            pltpu.sync_copy(src_hbm.at[pl.ds(off, C)], sidx)
            pltpu.sync_copy(dst_hbm.at[pl.ds(off, C)], didx)
            for i in range(C // 16):
                sidx[pl.ds(i * 16, 16)] = sidx[pl.ds(i * 16, 16)] + c * N
            pltpu.async_copy(nf2_hbm.at[sidx], rows, sem).wait()
            pltpu.sync_copy(ef_hbm.at[pl.ds(off, C), pl.ds(c * H, H)], efv)

            def row(e, carry2):
                for kk in range(H // 16):
                    sl = pl.ds(kk * 16, 16)
                    v = rows[e, sl] + efv[e, sl]
                    rows[e, sl] = jnp.maximum(v, 0.0)
                return carry2

            lax.fori_loop(0, C, row, 0)
            pltpu.sync_copy(rows, acc.at[didx], add=True)
            return carry

        lax.fori_loop(0, NCHUNK, chunk, 0)
        plsc.subcore_barrier()
        pltpu.sync_copy(acc.at[pl.ds(s * RPT, RPT)],
                        out_hbm.at[pl.ds(c * N + s * RPT, RPT)])

        @pl.when(s == 0)
        def _():
            pltpu.sync_copy(acc.at[pl.ds(NS * RPT, N - NS * RPT)],
                            out_hbm.at[pl.ds(c * N + NS * RPT, N - NS * RPT)])

    return k(nf2, src, dst, edge_feat)


_R = 400
_NB = N // _R


def _tc1_body(a_ref, b_ref, wa_ref, wb_ref, bias_ref, h_ref, stats_ref, acc_ref):
    i = pl.program_id(0)
    h = (jnp.dot(a_ref[...], wa_ref[...], preferred_element_type=jnp.float32)
         + jnp.dot(b_ref[...], wb_ref[...], preferred_element_type=jnp.float32)
         + bias_ref[...])
    h = jnp.where(h > 0, h, 0.01 * h)
    h_ref[...] = h

    @pl.when(i == 0)
    def _():
        acc_ref[...] = jnp.zeros_like(acc_ref)

    acc_ref[0:1, :] += jnp.sum(h, axis=0, keepdims=True)
    acc_ref[1:2, :] += jnp.sum(h * h, axis=0, keepdims=True)

    @pl.when(i == _NB - 1)
    def _():
        stats_ref[...] = acc_ref[...]


def _tc2_body(h_ref, stats_ref, gamma_ref, beta_ref, out_ref):
    mean = stats_ref[0:1, :] * (1.0 / N)
    var = stats_ref[1:2, :] * (1.0 / N) - mean * mean
    inv = lax.rsqrt(var + 1e-5)
    out_ref[...] = (h_ref[...] - mean) * (inv * gamma_ref[...]) + beta_ref[...]


def kernel(node_feat, edge_index, edge_feat, W, b, gamma, beta):
    src = edge_index[0]
    dst = edge_index[1]
    nf2 = node_feat.reshape(N, 2, H).transpose(1, 0, 2).reshape(2 * N, H)

    rst2 = _sc_aggregate(nf2, src, dst, edge_feat)
    a = rst2[:N]
    bb = rst2[N:]

    wa = W[:, :H].T
    wb = W[:, H:].T
    bias = b.reshape(1, D)

    h, stats = pl.pallas_call(
        _tc1_body,
        grid=(_NB,),
        in_specs=[
            pl.BlockSpec((_R, H), lambda i: (i, 0)),
            pl.BlockSpec((_R, H), lambda i: (i, 0)),
            pl.BlockSpec((H, D), lambda i: (0, 0)),
            pl.BlockSpec((H, D), lambda i: (0, 0)),
            pl.BlockSpec((1, D), lambda i: (0, 0)),
        ],
        out_specs=[
            pl.BlockSpec((_R, D), lambda i: (i, 0)),
            pl.BlockSpec((2, D), lambda i: (0, 0)),
        ],
        out_shape=[
            jax.ShapeDtypeStruct((N, D), jnp.float32),
            jax.ShapeDtypeStruct((2, D), jnp.float32),
        ],
        scratch_shapes=[pltpu.VMEM((2, D), jnp.float32)],
    )(a, bb, wa, wb, bias)

    out = pl.pallas_call(
        _tc2_body,
        grid=(_NB,),
        in_specs=[
            pl.BlockSpec((_R, D), lambda i: (i, 0)),
            pl.BlockSpec((2, D), lambda i: (0, 0)),
            pl.BlockSpec((1, D), lambda i: (0, 0)),
            pl.BlockSpec((1, D), lambda i: (0, 0)),
        ],
        out_specs=pl.BlockSpec((_R, D), lambda i: (i, 0)),
        out_shape=jax.ShapeDtypeStruct((N, D), jnp.float32),
    )(h, stats, gamma.reshape(1, D), beta.reshape(1, D))

    return out
